# Initial kernel scaffold; baseline (speedup 1.0000x reference)
import jax
import jax.numpy as jnp
from jax.experimental import pallas as pl


def kernel(x, wconv, bconv, whid, bhid, wneu, bneu):
    raise NotImplementedError("write your pallas kernel here")


# speedup vs baseline: 16.6103x; 16.6103x over previous
import functools

import jax
import jax.numpy as jnp
from jax.experimental import pallas as pl
from jax.experimental.pallas import tpu as pltpu


def _fused_kernel(x_ref, wtoe_ref, btoe_ref, wtail_ref, btail_ref,
                  whidT_ref, bhid_ref, wneuT_ref, bneu_ref, out_ref,
                  *, cin, m, p, s, nfull, ntail):
    B = x_ref.shape[1]
    wtoe = wtoe_ref[...]
    btoe = btoe_ref[...]
    stride = p * cin

    feat = jnp.zeros((p * m, B), jnp.float32)
    for c in range(nfull):
        xs = x_ref[pl.ds(c * stride, s), :]
        y = jnp.dot(wtoe, xs, preferred_element_type=jnp.float32) + btoe
        feat = jnp.maximum(feat, y)
    if ntail:
        xs = x_ref[pl.ds(nfull * stride, s), :]
        y = jnp.dot(wtail_ref[...], xs,
                    preferred_element_type=jnp.float32) + btail_ref[...]
        feat = jnp.maximum(feat, y)

    acc = feat[0:m, :]
    for q in range(1, p):
        acc = jnp.maximum(acc, feat[q * m:(q + 1) * m, :])

    h = jnp.dot(whidT_ref[...], acc,
                preferred_element_type=jnp.float32) + bhid_ref[...]
    h = jnp.maximum(h, 0.0)
    logit = jnp.dot(wneuT_ref[...], h,
                    preferred_element_type=jnp.float32) + bneu_ref[...]
    out_ref[...] = jax.nn.sigmoid(logit)


def kernel(x, wconv, bconv, whid, bhid, wneu, bneu, *, block_b=512):
    N, cin, L = x.shape
    M, _, K = wconv.shape
    H = whid.shape[1]
    lout = L - K + 1
    P = 128 // M
    S = ((P - 1) * cin + cin * K + 127) // 128 * 128
    nfull = lout // P
    ntail = lout - nfull * P
    nchunks = nfull + (1 if ntail else 0)
    R = (nchunks - 1) * P * cin + S

    npad = max(block_b, (N + block_b - 1) // block_b * block_b)

    x2 = jnp.transpose(x.astype(jnp.bfloat16), (2, 1, 0)).reshape(L * cin, N)
    x2 = jnp.pad(x2, ((0, R - L * cin), (0, npad - N)))

    wflat = jnp.transpose(wconv.astype(jnp.float32), (0, 2, 1)).reshape(M, K * cin)
    wtoe = jnp.zeros((P * M, S), jnp.float32)
    for q in range(P):
        wtoe = jax.lax.dynamic_update_slice(wtoe, wflat, (q * M, q * cin))
    btoe = jnp.tile(bconv.astype(jnp.float32).reshape(1, M), (P, 1)).reshape(P * M, 1)

    qvalid = (jnp.arange(P) < ntail).astype(jnp.float32)
    rowmask = jnp.repeat(qvalid, M).reshape(P * M, 1)
    wtail = (wtoe * rowmask).astype(jnp.bfloat16)
    btail = jnp.where(rowmask > 0, btoe, -1e30)
    wtoe = wtoe.astype(jnp.bfloat16)

    whidT = whid.T.astype(jnp.float32)
    bhid2 = bhid.reshape(H, 1).astype(jnp.float32)
    wneuT = wneu.T.astype(jnp.float32)
    bneu2 = bneu.reshape(1, 1).astype(jnp.float32)

    kfn = functools.partial(_fused_kernel, cin=cin, m=M, p=P, s=S,
                            nfull=nfull, ntail=ntail)
    out = pl.pallas_call(
        kfn,
        out_shape=jax.ShapeDtypeStruct((1, npad), jnp.float32),
        grid_spec=pltpu.PrefetchScalarGridSpec(
            num_scalar_prefetch=0,
            grid=(npad // block_b,),
            in_specs=[
                pl.BlockSpec((R, block_b), lambda n: (0, n)),
                pl.BlockSpec((P * M, S), lambda n: (0, 0)),
                pl.BlockSpec((P * M, 1), lambda n: (0, 0)),
                pl.BlockSpec((P * M, S), lambda n: (0, 0)),
                pl.BlockSpec((P * M, 1), lambda n: (0, 0)),
                pl.BlockSpec((H, M), lambda n: (0, 0)),
                pl.BlockSpec((H, 1), lambda n: (0, 0)),
                pl.BlockSpec((1, H), lambda n: (0, 0)),
                pl.BlockSpec((1, 1), lambda n: (0, 0)),
            ],
            out_specs=pl.BlockSpec((1, block_b), lambda n: (0, n)),
        ),
        compiler_params=pltpu.CompilerParams(
            dimension_semantics=("parallel",)),
    )(x2, wtoe, btoe, wtail, btail, whidT, bhid2, wneuT, bneu2)
    return out[0, :N].reshape(N, 1)


# speedup vs baseline: 20.4811x; 1.2330x over previous
import functools

import jax
import jax.numpy as jnp
from jax.experimental import pallas as pl
from jax.experimental.pallas import tpu as pltpu


def _fused_kernel(x_ref, wtoe_ref, wtail_ref, bconv_ref, whidT_ref, bhid_ref,
                  wneuT_ref, bneu_ref, out_ref,
                  *, cin, m, p, s, nfull, tail_start):
    B = x_ref.shape[1]
    wtoe = wtoe_ref[...]
    stride = p * cin

    feat = jnp.full((p * m, B), -1e30, jnp.float32)
    for c in range(nfull):
        xs = x_ref[pl.ds(c * stride, s), :]
        feat = jnp.maximum(feat, jnp.dot(wtoe, xs,
                                         preferred_element_type=jnp.float32))
    if tail_start is not None:
        xs = x_ref[pl.ds(tail_start, s), :]
        feat = jnp.maximum(feat, jnp.dot(wtail_ref[...], xs,
                                         preferred_element_type=jnp.float32))

    acc = feat[0:m, :]
    for q in range(1, p):
        acc = jnp.maximum(acc, feat[q * m:(q + 1) * m, :])
    acc = jnp.maximum(acc + bconv_ref[...], 0.0)

    h = jnp.dot(whidT_ref[...], acc,
                preferred_element_type=jnp.float32) + bhid_ref[...]
    h = jnp.maximum(h, 0.0)
    logit = jnp.dot(wneuT_ref[...], h,
                    preferred_element_type=jnp.float32) + bneu_ref[...]
    out_ref[...] = jax.nn.sigmoid(logit)


def _toeplitz(wflat, m, p, s, cin, shift):
    wt = jnp.zeros((p * m, s), jnp.float32)
    for q in range(p):
        wt = jax.lax.dynamic_update_slice(wt, wflat, (q * m, shift + q * cin))
    return wt.astype(jnp.bfloat16)


def kernel(x, wconv, bconv, whid, bhid, wneu, bneu, *, block_b=1024):
    N, cin, L = x.shape
    M, _, K = wconv.shape
    H = whid.shape[1]
    lout = L - K + 1
    P = 128 // M
    S = ((P - 1) * cin + cin * K + 127) // 128 * 128
    nfull = lout // P
    assert nfull >= 1
    ntail = lout - nfull * P

    if ntail:
        l0 = lout - P
        tail_start = l0 * cin // 16 * 16
        shift = l0 * cin - tail_start
        assert shift + (P - 1) * cin + K * cin <= S
        assert tail_start + S <= L * cin
    else:
        tail_start, shift = None, 0

    npad = max(block_b, (N + block_b - 1) // block_b * block_b)

    x2 = jnp.transpose(x.astype(jnp.bfloat16), (2, 1, 0)).reshape(L * cin, N)
    if npad != N:
        x2 = jnp.pad(x2, ((0, 0), (0, npad - N)))

    wflat = jnp.transpose(wconv.astype(jnp.float32), (0, 2, 1)).reshape(M, K * cin)
    wtoe = _toeplitz(wflat, M, P, S, cin, 0)
    wtail = _toeplitz(wflat, M, P, S, cin, shift) if ntail else wtoe

    bconv2 = bconv.reshape(M, 1).astype(jnp.float32)
    whidT = whid.T.astype(jnp.float32)
    bhid2 = bhid.reshape(H, 1).astype(jnp.float32)
    wneuT = wneu.T.astype(jnp.float32)
    bneu2 = bneu.reshape(1, 1).astype(jnp.float32)

    kfn = functools.partial(_fused_kernel, cin=cin, m=M, p=P, s=S,
                            nfull=nfull, tail_start=tail_start)
    out = pl.pallas_call(
        kfn,
        out_shape=jax.ShapeDtypeStruct((1, npad), jnp.float32),
        grid_spec=pltpu.PrefetchScalarGridSpec(
            num_scalar_prefetch=0,
            grid=(npad // block_b,),
            in_specs=[
                pl.BlockSpec((L * cin, block_b), lambda n: (0, n)),
                pl.BlockSpec((P * M, S), lambda n: (0, 0)),
                pl.BlockSpec((P * M, S), lambda n: (0, 0)),
                pl.BlockSpec((M, 1), lambda n: (0, 0)),
                pl.BlockSpec((H, M), lambda n: (0, 0)),
                pl.BlockSpec((H, 1), lambda n: (0, 0)),
                pl.BlockSpec((1, H), lambda n: (0, 0)),
                pl.BlockSpec((1, 1), lambda n: (0, 0)),
            ],
            out_specs=pl.BlockSpec((1, block_b), lambda n: (0, n)),
        ),
        compiler_params=pltpu.CompilerParams(
            dimension_semantics=("parallel",)),
    )(x2, wtoe, wtail, bconv2, whidT, bhid2, wneuT, bneu2)
    return out[0, :N].reshape(N, 1)
